# baseline (speedup 1.0000x reference)
import functools

import jax
import jax.numpy as jnp
from jax import lax
from jax.experimental import pallas as pl
from jax.experimental.pallas import tpu as pltpu
from jax.experimental.pallas import tpu_sc as plsc

NN = 100000
NP = 1600000
NW = 32
NPW = 3200
NGRP = NPW // 16
KU = 16
CH = 16384
SLAB = 4096
CROWS = CH // SLAB
SPAD = 3232
INT_MAX = 2**31 - 1


@functools.partial(
    pl.kernel,
    mesh=plsc.VectorSubcoreMesh(core_axis_name="c", subcore_axis_name="s"),
    compiler_params=pltpu.CompilerParams(needs_layout_passes=False),
    out_type=jax.ShapeDtypeStruct((NW * NPW,), jnp.int32),
    scratch_types=[
        pltpu.VMEM((SPAD,), jnp.int32),
        pltpu.VMEM((CH,), jnp.int32),
        pltpu.VMEM((CH,), jnp.int32),
        pltpu.VMEM((CH,), jnp.int32),
        pltpu.VMEM((CH,), jnp.int32),
        pltpu.VMEM((NPW,), jnp.int32),
        pltpu.SemaphoreType.DMA,
        pltpu.SemaphoreType.DMA,
        pltpu.SemaphoreType.DMA,
        pltpu.SemaphoreType.DMA,
    ],
)
def _segmin_kernel(start_hbm, flat_hbm, spm_hbm, out_hbm,
                   sbuf, fbufA, vbufA, fbufB, vbufB, obuf, semA, semB, semFA, semFB):
    cid = lax.axis_index("c")
    sid = lax.axis_index("s")
    w = sid * 2 + cid
    nbase = w * NPW
    pltpu.sync_copy(start_hbm.at[pl.ds(nbase, SPAD)], sbuf)

    lane = lax.iota(jnp.int32, 16)
    p0 = sbuf[pl.ds(0, 16)][0]
    p1 = sbuf[pl.ds(NPW, 16)][0]
    s0 = plsc.load_gather(sbuf, [lane])
    e0 = plsc.load_gather(sbuf, [lane + 1])
    cinit = (p0 // 8) * 8

    def flat_copy(c0, fbuf, semf):
        return pltpu.make_async_copy(
            flat_hbm.at[pl.ds(pl.multiple_of(c0, 8), CH)], fbuf, semf)

    def fire(fbuf, vbuf, sem):
        def one(r, x):
            pltpu.make_async_copy(
                spm_hbm.at[fbuf.at[pl.ds(r * SLAB, SLAB)]],
                vbuf.at[pl.ds(r * SLAB, SLAB)], sem).start()
            return x

        lax.fori_loop(0, CROWS, one, 0)

    def drain(fbuf, vbuf, sem):
        def one(r, x):
            pltpu.make_async_copy(
                spm_hbm.at[fbuf.at[pl.ds(r * SLAB, SLAB)]],
                vbuf.at[pl.ds(r * SLAB, SLAB)], sem).wait()
            return x

        lax.fori_loop(0, CROWS, one, 0)

    def walk(c0, fbuf, vbuf, carry):
        c1 = c0 + CH

        def in_cond(st2):
            go, gi2, e2, ptr2, av2, aq2 = st2
            return go & (gi2 < NGRP)

        def in_body(st2):
            go, gi2, e2, ptr2, av2, aq2 = st2
            limit = jnp.minimum(e2, c1)
            base = ptr2 - c0
            vs, qs = [], []
            for k in range(KU):
                q = ptr2 + k
                a = q < limit
                v = plsc.load_gather(vbuf, [base + k], mask=a)
                vs.append(jnp.where(a, v, INT_MAX))
                qs.append(q)
            while len(vs) > 1:
                nvs, nqs = [], []
                for i in range(0, len(vs), 2):
                    c = vs[i + 1] < vs[i]
                    nvs.append(jnp.where(c, vs[i + 1], vs[i]))
                    nqs.append(jnp.where(c, qs[i + 1], qs[i]))
                vs, qs = nvs, nqs
            upd = vs[0] < av2
            av3 = jnp.where(upd, vs[0], av2)
            aq3 = jnp.where(upd, qs[0], aq2)
            ptr3 = ptr2 + jnp.clip(limit - ptr2, 0, KU)
            done = ~jnp.any(ptr3 < e2)
            blocked = (~jnp.any((ptr3 < e2) & (ptr3 < c1))) & (~done)

            def emit_adv(args):
                gi_c, e_c, av_c, aq_c = args
                obuf[pl.ds(gi_c * 16, 16)] = aq_c
                gi_n = gi_c + 1
                base = gi_n * 16
                s_n = plsc.load_gather(sbuf, [base + lane])
                e_n = plsc.load_gather(sbuf, [base + lane + 1])
                return (gi_n, e_n, s_n,
                        jnp.full((16,), INT_MAX, jnp.int32),
                        jnp.full((16,), -1, jnp.int32))

            def stay(args):
                gi_c, e_c, av_c, aq_c = args
                return (gi_c, e_c, ptr3, av_c, aq_c)

            gi3, e3, ptr4, av4, aq4 = lax.cond(
                done, emit_adv, stay, (gi2, e2, av3, aq3))
            return (~blocked, gi3, e3, ptr4, av4, aq4)

        st = (jnp.bool_(True),) + carry
        st = lax.while_loop(in_cond, in_body, st)
        return st[1:]

    nchunk = lax.max((p1 - cinit + CH - 1) // CH, jnp.int32(1))
    npair = (nchunk + 1) // 2

    flat_copy(cinit, fbufA, semFA).start()
    flat_copy(cinit, fbufA, semFA).wait()
    fire(fbufA, vbufA, semA)

    @pl.when(cinit + CH < p1)
    def _():
        flat_copy(cinit + CH, fbufB, semFB).start()

    def pair_body(j, carry):
        ca = cinit + j * (2 * CH)
        cb = ca + CH

        @pl.when(cb < p1)
        def _():
            flat_copy(cb, fbufB, semFB).wait()
            fire(fbufB, vbufB, semB)

        drain(fbufA, vbufA, semA)

        @pl.when(ca + 2 * CH < p1)
        def _():
            flat_copy(ca + 2 * CH, fbufA, semFA).start()

        carry2 = walk(ca, fbufA, vbufA, carry)

        def odd_chunk(car):
            drain(fbufB, vbufB, semB)

            @pl.when(ca + 2 * CH < p1)
            def _():
                flat_copy(ca + 2 * CH, fbufA, semFA).wait()
                fire(fbufA, vbufA, semA)

            @pl.when(cb + 2 * CH < p1)
            def _():
                flat_copy(cb + 2 * CH, fbufB, semFB).start()

            return walk(cb, fbufB, vbufB, car)

        return lax.cond(cb < p1, odd_chunk, lambda car: car, carry2)

    carry0 = (jnp.int32(0), e0, s0,
              jnp.full((16,), INT_MAX, jnp.int32),
              jnp.full((16,), -1, jnp.int32))
    lax.fori_loop(0, npair, pair_body, carry0)

    def prep(t, x):
        off = t * 16
        q = obuf[pl.ds(off, 16)]
        dummy = nbase + off + lane
        fbufA[pl.ds(off, 16)] = jnp.where(q >= 0, q, dummy)
        return x

    lax.fori_loop(0, NPW // 16, prep, 0)

    def fire2(r, x):
        pltpu.make_async_copy(
            flat_hbm.at[fbufA.at[pl.ds(r * 640, 640)]],
            vbufA.at[pl.ds(r * 640, 640)], semA).start()
        return x

    lax.fori_loop(0, NPW // 640, fire2, 0)

    def drain2(r, x):
        pltpu.make_async_copy(
            flat_hbm.at[fbufA.at[pl.ds(r * 640, 640)]],
            vbufA.at[pl.ds(r * 640, 640)], semA).wait()
        return x

    lax.fori_loop(0, NPW // 640, drain2, 0)

    def comb(t, x):
        off = t * 16
        q = obuf[pl.ds(off, 16)]
        p = vbufA[pl.ds(off, 16)]
        obuf[pl.ds(off, 16)] = jnp.where(q >= 0, p, jnp.int32(0))
        return x

    lax.fori_loop(0, NPW // 16, comb, 0)

    pltpu.sync_copy(obuf, out_hbm.at[pl.ds(nbase, NPW)])


def kernel(flat_node2pin_start, flat_node2pin, sorted_pin_map):
    start_pad = jnp.pad(flat_node2pin_start,
                        (0, NW * NPW + SPAD - (NN + 1)),
                        constant_values=NP)
    flat_pad = jnp.pad(flat_node2pin, (0, 2 * CH + 8))
    out = _segmin_kernel(start_pad, flat_pad, sorted_pin_map)
    return out[:NN]


# speedup vs baseline: 1.2472x; 1.2472x over previous
import functools

import jax
import jax.numpy as jnp
from jax import lax
from jax.experimental import pallas as pl
from jax.experimental.pallas import tpu as pltpu
from jax.experimental.pallas import tpu_sc as plsc

NN = 100000
NP = 1600000
NW = 32
NPW = 3200
NGRP = NPW // 16
KU = 16
CH = 16384
SLAB = 1024
CROWS = CH // SLAB
SPAD = 3232
INT_MAX = 2**31 - 1


@functools.partial(
    pl.kernel,
    mesh=plsc.VectorSubcoreMesh(core_axis_name="c", subcore_axis_name="s"),
    compiler_params=pltpu.CompilerParams(needs_layout_passes=False),
    out_type=jax.ShapeDtypeStruct((NW * NPW,), jnp.int32),
    scratch_types=[
        pltpu.VMEM((SPAD,), jnp.int32),
        pltpu.VMEM((CH,), jnp.int32),
        pltpu.VMEM((CH,), jnp.int32),
        pltpu.VMEM((CH,), jnp.int32),
        pltpu.VMEM((CH,), jnp.int32),
        pltpu.VMEM((NPW,), jnp.int32),
        pltpu.SemaphoreType.DMA,
        pltpu.SemaphoreType.DMA,
        pltpu.SemaphoreType.DMA,
        pltpu.SemaphoreType.DMA,
    ],
)
def _segmin_kernel(start_hbm, flat_hbm, spm_hbm, out_hbm,
                   sbuf, fbufA, vbufA, fbufB, vbufB, obuf, semA, semB, semFA, semFB):
    cid = lax.axis_index("c")
    sid = lax.axis_index("s")
    w = sid * 2 + cid
    nbase = w * NPW
    pltpu.sync_copy(start_hbm.at[pl.ds(nbase, SPAD)], sbuf)

    lane = lax.iota(jnp.int32, 16)
    p0 = sbuf[pl.ds(0, 16)][0]
    p1 = sbuf[pl.ds(NPW, 16)][0]
    s0 = plsc.load_gather(sbuf, [lane])
    e0 = plsc.load_gather(sbuf, [lane + 1])
    cinit = (p0 // 8) * 8

    def flat_copy(c0, fbuf, semf):
        return pltpu.make_async_copy(
            flat_hbm.at[pl.ds(pl.multiple_of(c0, 8), CH)], fbuf, semf)

    def nrows(c0):
        return jnp.clip((p1 - c0 + SLAB - 1) // SLAB, 0, CROWS)

    def fire(c0, fbuf, vbuf, sem):
        def one(r, x):
            pltpu.make_async_copy(
                spm_hbm.at[fbuf.at[pl.ds(r * SLAB, SLAB)]],
                vbuf.at[pl.ds(r * SLAB, SLAB)], sem).start()
            return x

        lax.fori_loop(0, nrows(c0), one, 0)

    def drain(c0, fbuf, vbuf, sem):
        def one(r, x):
            pltpu.make_async_copy(
                spm_hbm.at[fbuf.at[pl.ds(r * SLAB, SLAB)]],
                vbuf.at[pl.ds(r * SLAB, SLAB)], sem).wait()
            return x

        lax.fori_loop(0, nrows(c0), one, 0)

    def walk(c0, fbuf, vbuf, carry):
        c1 = c0 + CH

        def in_cond(st2):
            go, gi2, e2, ptr2, av2, aq2 = st2
            return go & (gi2 < NGRP)

        def in_body(st2):
            go, gi2, e2, ptr2, av2, aq2 = st2
            limit = jnp.minimum(e2, c1)
            base = ptr2 - c0
            vs, qs = [], []
            for k in range(KU):
                q = ptr2 + k
                a = q < limit
                v = plsc.load_gather(vbuf, [base + k], mask=a)
                vs.append(jnp.where(a, v, INT_MAX))
                qs.append(q)
            while len(vs) > 1:
                nvs, nqs = [], []
                for i in range(0, len(vs), 2):
                    c = vs[i + 1] < vs[i]
                    nvs.append(jnp.where(c, vs[i + 1], vs[i]))
                    nqs.append(jnp.where(c, qs[i + 1], qs[i]))
                vs, qs = nvs, nqs
            upd = vs[0] < av2
            av3 = jnp.where(upd, vs[0], av2)
            aq3 = jnp.where(upd, qs[0], aq2)
            ptr3 = ptr2 + jnp.clip(limit - ptr2, 0, KU)
            done = ~jnp.any(ptr3 < e2)
            blocked = (~jnp.any((ptr3 < e2) & (ptr3 < c1))) & (~done)

            def emit_adv(args):
                gi_c, e_c, av_c, aq_c = args
                obuf[pl.ds(gi_c * 16, 16)] = aq_c
                gi_n = gi_c + 1
                base = gi_n * 16
                s_n = plsc.load_gather(sbuf, [base + lane])
                e_n = plsc.load_gather(sbuf, [base + lane + 1])
                return (gi_n, e_n, s_n,
                        jnp.full((16,), INT_MAX, jnp.int32),
                        jnp.full((16,), -1, jnp.int32))

            def stay(args):
                gi_c, e_c, av_c, aq_c = args
                return (gi_c, e_c, ptr3, av_c, aq_c)

            gi3, e3, ptr4, av4, aq4 = lax.cond(
                done, emit_adv, stay, (gi2, e2, av3, aq3))
            return (~blocked, gi3, e3, ptr4, av4, aq4)

        st = (jnp.bool_(True),) + carry
        st = lax.while_loop(in_cond, in_body, st)
        return st[1:]

    nchunk = lax.max((p1 - cinit + CH - 1) // CH, jnp.int32(1))
    npair = (nchunk + 1) // 2

    flat_copy(cinit, fbufA, semFA).start()
    flat_copy(cinit, fbufA, semFA).wait()
    fire(cinit, fbufA, vbufA, semA)

    @pl.when(cinit + CH < p1)
    def _():
        flat_copy(cinit + CH, fbufB, semFB).start()

    def pair_body(j, carry):
        ca = cinit + j * (2 * CH)
        cb = ca + CH

        @pl.when(cb < p1)
        def _():
            flat_copy(cb, fbufB, semFB).wait()
            fire(cb, fbufB, vbufB, semB)

        drain(ca, fbufA, vbufA, semA)

        @pl.when(ca + 2 * CH < p1)
        def _():
            flat_copy(ca + 2 * CH, fbufA, semFA).start()

        carry2 = walk(ca, fbufA, vbufA, carry)

        def odd_chunk(car):
            drain(cb, fbufB, vbufB, semB)

            @pl.when(ca + 2 * CH < p1)
            def _():
                flat_copy(ca + 2 * CH, fbufA, semFA).wait()
                fire(ca + 2 * CH, fbufA, vbufA, semA)

            @pl.when(cb + 2 * CH < p1)
            def _():
                flat_copy(cb + 2 * CH, fbufB, semFB).start()

            return walk(cb, fbufB, vbufB, car)

        return lax.cond(cb < p1, odd_chunk, lambda car: car, carry2)

    carry0 = (jnp.int32(0), e0, s0,
              jnp.full((16,), INT_MAX, jnp.int32),
              jnp.full((16,), -1, jnp.int32))
    lax.fori_loop(0, npair, pair_body, carry0)

    def prep(t, x):
        off = t * 16
        q = obuf[pl.ds(off, 16)]
        dummy = nbase + off + lane
        fbufA[pl.ds(off, 16)] = jnp.where(q >= 0, q, dummy)
        return x

    lax.fori_loop(0, NPW // 16, prep, 0)

    def fire2(r, x):
        pltpu.make_async_copy(
            flat_hbm.at[fbufA.at[pl.ds(r * 640, 640)]],
            vbufA.at[pl.ds(r * 640, 640)], semA).start()
        return x

    lax.fori_loop(0, NPW // 640, fire2, 0)

    def drain2(r, x):
        pltpu.make_async_copy(
            flat_hbm.at[fbufA.at[pl.ds(r * 640, 640)]],
            vbufA.at[pl.ds(r * 640, 640)], semA).wait()
        return x

    lax.fori_loop(0, NPW // 640, drain2, 0)

    def comb(t, x):
        off = t * 16
        q = obuf[pl.ds(off, 16)]
        p = vbufA[pl.ds(off, 16)]
        obuf[pl.ds(off, 16)] = jnp.where(q >= 0, p, jnp.int32(0))
        return x

    lax.fori_loop(0, NPW // 16, comb, 0)

    pltpu.sync_copy(obuf, out_hbm.at[pl.ds(nbase, NPW)])


def kernel(flat_node2pin_start, flat_node2pin, sorted_pin_map):
    start_pad = jnp.pad(flat_node2pin_start,
                        (0, NW * NPW + SPAD - (NN + 1)),
                        constant_values=NP)
    flat_pad = jnp.pad(flat_node2pin, (0, 2 * CH + 8))
    out = _segmin_kernel(start_pad, flat_pad, sorted_pin_map)
    return out[:NN]
